# Initial kernel scaffold; baseline (speedup 1.0000x reference)
import jax
import jax.numpy as jnp
from jax.experimental import pallas as pl


def kernel(x, edge_index, W_lin, att, bias, W1, b1, W2, b2):
    raise NotImplementedError("write your pallas kernel here")


# speedup vs baseline: 132.8137x; 132.8137x over previous
import functools

import jax
import jax.numpy as jnp
from jax import lax
from jax.experimental import pallas as pl
from jax.experimental.pallas import tpu as pltpu
from jax.experimental.pallas import tpu_sc as plsc

_NC = 2
_NS = 16
_NW = _NC * _NS
_L = 16


def _presence_counts(row, n_nodes):
    e = row.shape[0]
    epw = e // _NW
    mesh = plsc.VectorSubcoreMesh(core_axis_name="c", subcore_axis_name="s")

    @functools.partial(
        pl.kernel,
        out_type=jax.ShapeDtypeStruct((_NW * n_nodes,), jnp.float32),
        mesh=mesh,
        scratch_types=[
            pltpu.VMEM((epw,), jnp.int32),
            pltpu.VMEM((n_nodes,), jnp.float32),
        ],
        compiler_params=pltpu.CompilerParams(needs_layout_passes=False),
    )
    def sc_kernel(row_hbm, out_hbm, idx_v, mask_v):
        cid = lax.axis_index("c")
        sid = lax.axis_index("s")
        wid = sid * _NC + cid

        def zero_body(i, carry):
            mask_v[pl.ds(i * _L, _L)] = jnp.zeros((_L,), jnp.float32)
            return carry

        lax.fori_loop(0, n_nodes // _L, zero_body, 0, unroll=4)

        pltpu.sync_copy(row_hbm.at[pl.ds(wid * epw, epw)], idx_v)

        ones = jnp.ones((_L,), jnp.float32)

        def scat_body(i, carry):
            idx = idx_v[pl.ds(i * _L, _L)]
            plsc.store_scatter(mask_v, [idx], ones)
            return carry

        lax.fori_loop(0, epw // _L, scat_body, 0, unroll=4)

        pltpu.sync_copy(mask_v, out_hbm.at[pl.ds(wid * n_nodes, n_nodes)])

    return sc_kernel(row)


def _tc_body(x_ref, wl_ref, w1_ref, b1_ref, w2_ref, b2_ref, bias_ref, cnt_ref,
             o_ref):
    xb = x_ref[...]
    dn = (((1,), (1,)), ((), ()))
    xp = lax.dot_general(xb, wl_ref[...], dn,
                         preferred_element_type=jnp.float32)
    hh = lax.dot_general(xp, w1_ref[...], dn,
                         preferred_element_type=jnp.float32)
    hh = hh + b1_ref[...]
    hh = 0.5 * hh * (1.0 + lax.erf(hh * (2.0**-0.5)))
    pw = jnp.sum(hh * w2_ref[...], axis=1, keepdims=True)
    pw = jax.nn.sigmoid(pw + b2_ref[0, 0])
    present = jnp.sum(cnt_ref[...], axis=1, keepdims=True) > 0.0
    scale = jnp.where(present, 1.0 + pw, 0.0)
    o_ref[...] = xp * scale + bias_ref[...]


def kernel(x, edge_index, W_lin, att, bias, W1, b1, W2, b2):
    del att
    n, d_in = x.shape
    d_out = W_lin.shape[0]
    h = W1.shape[0]

    row = edge_index[0]
    cnt = _presence_counts(row, n)
    cnt_t = cnt.reshape(_NW, n).T

    blk = 1000
    grid = n // blk

    out = pl.pallas_call(
        _tc_body,
        grid=(grid,),
        in_specs=[
            pl.BlockSpec((blk, d_in), lambda i: (i, 0)),
            pl.BlockSpec((d_out, d_in), lambda i: (0, 0)),
            pl.BlockSpec((h, d_in), lambda i: (0, 0)),
            pl.BlockSpec((1, h), lambda i: (0, 0)),
            pl.BlockSpec((1, h), lambda i: (0, 0)),
            pl.BlockSpec((1, 1), lambda i: (0, 0)),
            pl.BlockSpec((1, d_out), lambda i: (0, 0)),
            pl.BlockSpec((blk, _NW), lambda i: (i, 0)),
        ],
        out_specs=pl.BlockSpec((blk, d_out), lambda i: (i, 0)),
        out_shape=jax.ShapeDtypeStruct((n, d_out), jnp.float32),
    )(x, W_lin, W1, b1.reshape(1, h), W2, b2.reshape(1, 1),
      bias.reshape(1, d_out), cnt_t)
    return out
